# baseline (speedup 1.0000x reference)
import jax
import jax.numpy as jnp
from jax import lax
from jax.experimental import pallas as pl
from jax.experimental.pallas import tpu as pltpu
from jax.experimental.pallas import tpu_sc as plsc

DIMS = (8, 16, 32, 64, 128)
OFFS = (0, 8, 24, 56)
B = 16384
NC, NS = 2, 16
NW = NC * NS
BPW = B // NW
NCH = BPW // 128
ROW_BLK = 2048


def _sc_gather_1t_body(x_hbm, tab, out, idx_v, rb, sem):
    wid = lax.axis_index("s") * NC + lax.axis_index("c")
    base = wid * BPW
    pltpu.sync_copy(x_hbm.at[pl.ds(wid * NCH, NCH)], idx_v)
    copies = [pltpu.async_copy(tab.at[idx_v.at[j]],
                               rb.at[pl.ds(j * 128, 128)], sem)
              for j in range(NCH)]
    for c in copies:
        c.wait()
    pltpu.sync_copy(rb, out.at[pl.ds(base, BPW)])


def _sc_gather_1t(x2d, tab, d):
    mesh = plsc.VectorSubcoreMesh(core_axis_name="c", subcore_axis_name="s")
    return pl.kernel(
        _sc_gather_1t_body,
        out_type=jax.ShapeDtypeStruct((B, d), jnp.float32),
        mesh=mesh,
        scratch_types=[
            pltpu.VMEM((NCH, 128), jnp.int32),
            pltpu.VMEM((BPW, d), jnp.float32),
            pltpu.SemaphoreType.DMA,
        ],
        compiler_params=pltpu.CompilerParams(use_tc_tiling_on_sc=False),
    )(x2d, tab)


def _sc_gather_e4_body(x_hbm, e4, o4, idx_v, ra, rb, rc, rd, sem):
    wid = lax.axis_index("s") * NC + lax.axis_index("c")
    base = wid * BPW
    pltpu.sync_copy(x_hbm.at[pl.ds(wid * NCH, NCH)], idx_v)
    bufs = (ra, rb, rc, rd)
    copies = [pltpu.async_copy(e4.at[idx_v.at[j]], bufs[j], sem)
              for j in range(NCH)]
    for c in copies:
        c.wait()
    outs = [pltpu.async_copy(bufs[j], o4.at[pl.ds(base + j * 128, 128)], sem)
            for j in range(NCH)]
    for c in outs:
        c.wait()


def _sc_gather_e4(x2d, e4):
    mesh = plsc.VectorSubcoreMesh(core_axis_name="c", subcore_axis_name="s")
    return pl.kernel(
        _sc_gather_e4_body,
        out_type=jax.ShapeDtypeStruct((B, 128), jnp.float32),
        mesh=mesh,
        scratch_types=[pltpu.VMEM((NCH, 128), jnp.int32)]
        + [pltpu.VMEM((128, 128), jnp.float32)] * 4
        + [pltpu.SemaphoreType.DMA],
    )(x2d, e4)


def _tc_combine_body(dw_ref, g0, g1, g2, g3, e4_ref,
                     pw0, pw1, pw2, pw3, bstack_ref, out_ref):
    w = [dw_ref[i] for i in range(5)]
    dn = (((1,), (1,)), ((), ()))
    acc = w[4] * e4_ref[...]
    for wi, g, pw in zip(w, (g0, g1, g2, g3), (pw0, pw1, pw2, pw3)):
        acc += wi * lax.dot_general(g[...], pw[...], dn,
                                    preferred_element_type=jnp.float32)
    bias = (w[0] * bstack_ref[0, :] + w[1] * bstack_ref[1, :]
            + w[2] * bstack_ref[2, :] + w[3] * bstack_ref[3, :])
    out_ref[...] = acc + bias[None, :]


def _tc_combine(gs, e4, pws, bstack, dw):
    return pl.pallas_call(
        _tc_combine_body,
        grid=(B // ROW_BLK,),
        in_specs=[pl.BlockSpec(memory_space=pltpu.SMEM)]
        + [pl.BlockSpec((ROW_BLK, d), lambda i: (i, 0)) for d in DIMS]
        + [pl.BlockSpec((128, d), lambda i: (0, 0)) for d in DIMS[:-1]]
        + [pl.BlockSpec((4, 128), lambda i: (0, 0))],
        out_specs=pl.BlockSpec((ROW_BLK, 128), lambda i: (i, 0)),
        out_shape=jax.ShapeDtypeStruct((B, 128), jnp.float32),
    )(dw, *gs, e4, *pws, bstack)


def kernel(x, emb_0, emb_1, emb_2, emb_3, emb_4,
           proj_w_0, proj_b_0, proj_w_1, proj_b_1,
           proj_w_2, proj_b_2, proj_w_3, proj_b_3,
           dim_logits):
    dim_weights = jax.nn.softmax(dim_logits, axis=-1)
    x2d = x.astype(jnp.int32).reshape(B // 128, 128)
    gs = tuple(_sc_gather_1t(x2d, t, d)
               for t, d in zip((emb_0, emb_1, emb_2, emb_3), DIMS))
    e4 = _sc_gather_e4(x2d, emb_4)
    bstack = jnp.stack([proj_b_0, proj_b_1, proj_b_2, proj_b_3], axis=0)
    out = _tc_combine(gs, e4, (proj_w_0, proj_w_1, proj_w_2, proj_w_3),
                      bstack, dim_weights)
    return (out, dim_weights)


# speedup vs baseline: 1.1391x; 1.1391x over previous
import jax
import jax.numpy as jnp
from jax import lax
from jax.experimental import pallas as pl
from jax.experimental.pallas import tpu as pltpu
from jax.experimental.pallas import tpu_sc as plsc

DIMS = (8, 16, 32, 64, 128)
OFFS = (0, 8, 24, 56)
B = 16384
NC, NS = 2, 16
NW = NC * NS
BPW = B // NW
NCH = BPW // 128
ROW_BLK = 2048


def _sc_gather_body(x_hbm, e0, e1, e2, e3,
                    ocat,
                    idx_v, r0, r1, r2, r3, sem):
    wid = lax.axis_index("s") * NC + lax.axis_index("c")
    base = wid * BPW
    pltpu.sync_copy(x_hbm.at[pl.ds(wid * NCH, NCH)], idx_v)
    bufs = (r0, r1, r2, r3)
    copies = []
    for t, rb in zip((e0, e1, e2, e3), bufs):
        for j in range(NCH):
            copies.append(pltpu.async_copy(
                t.at[idx_v.at[j]], rb.at[pl.ds(j * 128, 128)], sem))
    for c in copies:
        c.wait()
    outs = []
    for rb, off, d in zip(bufs, OFFS, DIMS):
        outs.append(pltpu.async_copy(
            rb, ocat.at[pl.ds(base, BPW), pl.ds(off, d)], sem))
    for c in outs:
        c.wait()


def _sc_gather(x2d, e0, e1, e2, e3):
    mesh = plsc.VectorSubcoreMesh(core_axis_name="c", subcore_axis_name="s")
    return pl.kernel(
        _sc_gather_body,
        out_type=jax.ShapeDtypeStruct((B, 128), jnp.float32),
        mesh=mesh,
        scratch_types=[
            pltpu.VMEM((NCH, 128), jnp.int32),
            pltpu.VMEM((BPW, 8), jnp.float32),
            pltpu.VMEM((BPW, 16), jnp.float32),
            pltpu.VMEM((BPW, 32), jnp.float32),
            pltpu.VMEM((BPW, 64), jnp.float32),
            pltpu.SemaphoreType.DMA,
        ],
        compiler_params=pltpu.CompilerParams(use_tc_tiling_on_sc=False),
    )(x2d, e0, e1, e2, e3)


def _sc_gather_e4_body(x_hbm, e4, o4, idx_v, ra, rb, rc, rd, sem):
    wid = lax.axis_index("s") * NC + lax.axis_index("c")
    base = wid * BPW
    pltpu.sync_copy(x_hbm.at[pl.ds(wid * NCH, NCH)], idx_v)
    bufs = (ra, rb, rc, rd)
    copies = [pltpu.async_copy(e4.at[idx_v.at[j]], bufs[j], sem)
              for j in range(NCH)]
    for c in copies:
        c.wait()
    outs = [pltpu.async_copy(bufs[j], o4.at[pl.ds(base + j * 128, 128)], sem)
            for j in range(NCH)]
    for c in outs:
        c.wait()


def _sc_gather_e4(x2d, e4):
    mesh = plsc.VectorSubcoreMesh(core_axis_name="c", subcore_axis_name="s")
    return pl.kernel(
        _sc_gather_e4_body,
        out_type=jax.ShapeDtypeStruct((B, 128), jnp.float32),
        mesh=mesh,
        scratch_types=[pltpu.VMEM((NCH, 128), jnp.int32)]
        + [pltpu.VMEM((128, 128), jnp.float32)] * 4
        + [pltpu.SemaphoreType.DMA],
    )(x2d, e4)


def _tc_combine_body(dw_ref, cat_ref, e4_ref, pcat_ref, bstack_ref, out_ref):
    w = [dw_ref[i] for i in range(5)]
    cat = cat_ref[...]
    col = lax.broadcasted_iota(jnp.int32, (ROW_BLK, 128), 1)
    cat = jnp.where(col < 120, cat, 0.0)
    r = lax.broadcasted_iota(jnp.int32, (128, 128), 0)
    scale = jnp.where(r < 8, w[0],
            jnp.where(r < 24, w[1],
            jnp.where(r < 56, w[2],
            jnp.where(r < 120, w[3], 0.0))))
    p = pcat_ref[...] * scale
    acc = jnp.dot(cat, p, preferred_element_type=jnp.float32)
    bias = (w[0] * bstack_ref[0, :] + w[1] * bstack_ref[1, :]
            + w[2] * bstack_ref[2, :] + w[3] * bstack_ref[3, :])
    out_ref[...] = acc + bias[None, :] + w[4] * e4_ref[...]


def _tc_combine(cat, e4, pcat, bstack, dw):
    return pl.pallas_call(
        _tc_combine_body,
        grid=(B // ROW_BLK,),
        in_specs=[
            pl.BlockSpec(memory_space=pltpu.SMEM),
            pl.BlockSpec((ROW_BLK, 128), lambda i: (i, 0)),
            pl.BlockSpec((ROW_BLK, 128), lambda i: (i, 0)),
            pl.BlockSpec((128, 128), lambda i: (0, 0)),
            pl.BlockSpec((4, 128), lambda i: (0, 0)),
        ],
        out_specs=pl.BlockSpec((ROW_BLK, 128), lambda i: (i, 0)),
        out_shape=jax.ShapeDtypeStruct((B, 128), jnp.float32),
    )(dw, cat, e4, pcat, bstack)


def kernel(x, emb_0, emb_1, emb_2, emb_3, emb_4,
           proj_w_0, proj_b_0, proj_w_1, proj_b_1,
           proj_w_2, proj_b_2, proj_w_3, proj_b_3,
           dim_logits):
    dim_weights = jax.nn.softmax(dim_logits, axis=-1)
    x2d = x.astype(jnp.int32).reshape(B // 128, 128)
    cat = _sc_gather(x2d, emb_0, emb_1, emb_2, emb_3)
    e4 = _sc_gather_e4(x2d, emb_4)
    pcat = jnp.concatenate(
        [proj_w_0.T, proj_w_1.T, proj_w_2.T, proj_w_3.T,
         jnp.zeros((8, 128), jnp.float32)], axis=0)
    bstack = jnp.stack([proj_b_0, proj_b_1, proj_b_2, proj_b_3], axis=0)
    out = _tc_combine(cat, e4, pcat, bstack, dim_weights)
    return (out, dim_weights)
